# Initial kernel scaffold; baseline (speedup 1.0000x reference)
import functools

import jax
import jax.numpy as jnp
from jax import lax
from jax.experimental import pallas as pl
from jax.experimental.pallas import tpu as pltpu
from jax.experimental.pallas import tpu_sc as plsc

L = 100
E = 64
LE = L * E
LANES = 16
ROWS_PER_CHUNK = 4


@functools.lru_cache(maxsize=None)
def _make_sc_kernel(B: int):
    info = plsc.get_sparse_core_info()
    nw = info.num_cores * info.num_subcores
    rows_per_w = B // nw
    R = ROWS_PER_CHUNK
    n_chunks = rows_per_w // R
    mesh = plsc.VectorSubcoreMesh(core_axis_name="c", subcore_axis_name="s")

    @functools.partial(
        pl.kernel,
        mesh=mesh,
        out_type=jax.ShapeDtypeStruct((B * LE,), jnp.float32),
        scratch_types=[
            pltpu.VMEM((LE,), jnp.float32),
            pltpu.VMEM((R * L,), jnp.float32),
            pltpu.VMEM((R * LE,), jnp.float32),
        ],
    )
    def sc_kernel(in_hbm, emb_hbm, out_hbm, emb_v, in_v, out_v):
        c = lax.axis_index("c")
        s = lax.axis_index("s")
        wid = s * info.num_cores + c
        row0 = wid * rows_per_w
        pltpu.sync_copy(emb_hbm, emb_v)

        def chunk_body(g, carry):
            r0 = row0 + g * R
            pltpu.sync_copy(in_hbm.at[pl.ds(r0 * L, R * L)], in_v)
            for r in range(R):
                def i_body(i, c2):
                    idx = jnp.full((LANES,), r * L, jnp.int32) + i
                    sval = plsc.load_gather(in_v, [idx])
                    base = r * LE + i * E
                    for kk in range(E // LANES):
                        o = base + kk * LANES
                        out_v[pl.ds(o, LANES)] = (
                            sval * emb_v[pl.ds(i * E + kk * LANES, LANES)]
                        )
                    return c2
                lax.fori_loop(0, L, i_body, 0)
            pltpu.sync_copy(out_v, out_hbm.at[pl.ds(r0 * LE, R * LE)])
            return carry

        lax.fori_loop(0, n_chunks, chunk_body, 0)

    return sc_kernel


def kernel(input, emb_table):
    B = input.shape[0]
    out = _make_sc_kernel(B)(input.reshape(-1), emb_table.reshape(-1))
    return out.reshape(B, 1, LE)


# speedup vs baseline: 5.1450x; 5.1450x over previous
import functools

import jax
import jax.numpy as jnp
from jax import lax
from jax.experimental import pallas as pl
from jax.experimental.pallas import tpu as pltpu
from jax.experimental.pallas import tpu_sc as plsc

L = 100
E = 64
LE = L * E
LANES = 16
ROWS_PER_CHUNK = 4


@functools.lru_cache(maxsize=None)
def _make_sc_kernel(B: int):
    info = plsc.get_sparse_core_info()
    nw = info.num_cores * info.num_subcores
    rows_per_w = B // nw
    R = ROWS_PER_CHUNK
    n_chunks = rows_per_w // R
    mesh = plsc.VectorSubcoreMesh(core_axis_name="c", subcore_axis_name="s")

    @functools.partial(
        pl.kernel,
        mesh=mesh,
        out_type=jax.ShapeDtypeStruct((B * LE,), jnp.float32),
        scratch_types=[
            pltpu.VMEM((LE,), jnp.float32),
            pltpu.VMEM((R * L,), jnp.float32),
            pltpu.VMEM((R * LE,), jnp.float32),
        ],
        compiler_params=pltpu.CompilerParams(needs_layout_passes=False),
    )
    def sc_kernel(in_hbm, emb_hbm, out_hbm, emb_v, in_v, out_v):
        c = lax.axis_index("c")
        s = lax.axis_index("s")
        wid = s * info.num_cores + c
        row0 = wid * rows_per_w
        pltpu.sync_copy(emb_hbm, emb_v)

        def chunk_body(g, carry):
            r0 = row0 + g * R
            pltpu.sync_copy(in_hbm.at[pl.ds(r0 * L, R * L)], in_v)
            for r in range(R):
                def i_body(i, c2):
                    idx = jnp.full((LANES,), r * L, jnp.int32) + i
                    sval = plsc.load_gather(in_v, [idx])
                    base = r * LE + i * E
                    for kk in range(E // LANES):
                        o = base + kk * LANES
                        out_v[pl.ds(o, LANES)] = (
                            sval * emb_v[pl.ds(i * E + kk * LANES, LANES)]
                        )
                    return c2
                lax.fori_loop(0, L, i_body, 0)
            pltpu.sync_copy(out_v, out_hbm.at[pl.ds(r0 * LE, R * LE)])
            return carry

        lax.fori_loop(0, n_chunks, chunk_body, 0)

    return sc_kernel


def kernel(input, emb_table):
    B = input.shape[0]
    out = _make_sc_kernel(B)(input.reshape(-1), emb_table.reshape(-1))
    return out.reshape(B, 1, LE)


# speedup vs baseline: 26.9947x; 5.2468x over previous
import functools

import jax
import jax.numpy as jnp
from jax import lax
from jax.experimental import pallas as pl
from jax.experimental.pallas import tpu as pltpu
from jax.experimental.pallas import tpu_sc as plsc

L = 100
E = 64
LE = L * E
LANES = 16
ROWS_PER_CHUNK = 8


@functools.lru_cache(maxsize=None)
def _make_sc_kernel(B: int):
    info = plsc.get_sparse_core_info()
    nw = info.num_cores * info.num_subcores
    rows_per_w = B // nw
    R = ROWS_PER_CHUNK
    n_chunks = rows_per_w // R
    mesh = plsc.VectorSubcoreMesh(core_axis_name="c", subcore_axis_name="s")

    @functools.partial(
        pl.kernel,
        mesh=mesh,
        out_type=jax.ShapeDtypeStruct((B * LE,), jnp.float32),
        scratch_types=[
            pltpu.VMEM((LE,), jnp.float32),
            pltpu.VMEM((rows_per_w * L,), jnp.float32),
            pltpu.VMEM((R * LE,), jnp.float32),
            pltpu.VMEM((R * LE,), jnp.float32),
            pltpu.SemaphoreType.DMA,
            pltpu.SemaphoreType.DMA,
        ],
        compiler_params=pltpu.CompilerParams(needs_layout_passes=False),
    )
    def sc_kernel(in_hbm, emb_hbm, out_hbm, emb_v, in_v, out0, out1, sem0, sem1):
        c = lax.axis_index("c")
        s = lax.axis_index("s")
        wid = s * info.num_cores + c
        row0 = wid * rows_per_w
        pltpu.sync_copy(emb_hbm, emb_v)
        pltpu.sync_copy(in_hbm.at[pl.ds(row0 * L, rows_per_w * L)], in_v)

        outs = (out0, out1)
        sems = (sem0, sem1)

        def compute(g, out_v):
            @plsc.parallel_loop(0, L, unroll=4)
            def _(i):
                evs = [
                    emb_v[pl.ds(i * E + kk * LANES, LANES)]
                    for kk in range(E // LANES)
                ]
                for r in range(R):
                    idx = jnp.full((LANES,), (g * R + r) * L, jnp.int32) + i
                    sval = plsc.load_gather(in_v, [idx])
                    for kk in range(E // LANES):
                        out_v[pl.ds(r * LE + i * E + kk * LANES, LANES)] = (
                            sval * evs[kk]
                        )

        def start_store(g, b):
            pltpu.async_copy(
                outs[b], out_hbm.at[pl.ds((row0 + g * R) * LE, R * LE)], sems[b]
            )

        def wait_store(g, b):
            pltpu.make_async_copy(
                outs[b], out_hbm.at[pl.ds((row0 + g * R) * LE, R * LE)], sems[b]
            ).wait()

        for b in range(2):
            compute(jnp.int32(b), outs[b])
            start_store(jnp.int32(b), b)

        @pl.loop(2, n_chunks, step=2)
        def _(g):
            for b in range(2):
                wait_store(g + b - 2, b)
                compute(g + b, outs[b])
                start_store(g + b, b)

        for b in range(2):
            wait_store(jnp.int32(n_chunks - 2 + b), b)

    return sc_kernel


def kernel(input, emb_table):
    B = input.shape[0]
    out = _make_sc_kernel(B)(input.reshape(-1), emb_table.reshape(-1))
    return out.reshape(B, 1, LE)


# speedup vs baseline: 28.5212x; 1.0565x over previous
import functools

import jax
import jax.numpy as jnp
from jax import lax
from jax.experimental import pallas as pl
from jax.experimental.pallas import tpu as pltpu
from jax.experimental.pallas import tpu_sc as plsc

L = 100
E = 64
LE = L * E
LANES = 16
ROWS_PER_CHUNK = 8


@functools.lru_cache(maxsize=None)
def _make_sc_kernel(B: int):
    info = plsc.get_sparse_core_info()
    nw = info.num_cores * info.num_subcores
    rows_per_w = B // nw
    R = ROWS_PER_CHUNK
    n_chunks = rows_per_w // R
    mesh = plsc.VectorSubcoreMesh(core_axis_name="c", subcore_axis_name="s")

    @functools.partial(
        pl.kernel,
        mesh=mesh,
        out_type=jax.ShapeDtypeStruct((B * LE,), jnp.float32),
        scratch_types=[
            pltpu.VMEM((LE,), jnp.float32),
            pltpu.VMEM((rows_per_w, L), jnp.float32),
            pltpu.VMEM((R * LE,), jnp.float32),
            pltpu.VMEM((R * LE,), jnp.float32),
            pltpu.SemaphoreType.DMA,
            pltpu.SemaphoreType.DMA,
        ],
        compiler_params=pltpu.CompilerParams(needs_layout_passes=False),
    )
    def sc_kernel(in_hbm, emb_hbm, out_hbm, emb_v, in_v, out0, out1, sem0, sem1):
        c = lax.axis_index("c")
        s = lax.axis_index("s")
        wid = s * info.num_cores + c
        row0 = wid * rows_per_w
        pltpu.sync_copy(emb_hbm, emb_v)
        pltpu.sync_copy(in_hbm.at[pl.ds(row0, rows_per_w)], in_v)

        outs = (out0, out1)
        sems = (sem0, sem1)

        def compute(g, out_v):
            @plsc.parallel_loop(0, L, unroll=4)
            def _(i):
                evs = [
                    emb_v[pl.ds(i * E + kk * LANES, LANES)]
                    for kk in range(E // LANES)
                ]
                for r in range(R):
                    ridx = jnp.full((LANES,), g * R + r, jnp.int32)
                    iidx = jnp.full((LANES,), 0, jnp.int32) + i
                    sval = plsc.load_gather(in_v, [ridx, iidx])
                    for kk in range(E // LANES):
                        out_v[pl.ds(r * LE + i * E + kk * LANES, LANES)] = (
                            sval * evs[kk]
                        )

        def start_store(g, b):
            pltpu.async_copy(
                outs[b], out_hbm.at[pl.ds((row0 + g * R) * LE, R * LE)], sems[b]
            )

        def wait_store(g, b):
            pltpu.make_async_copy(
                outs[b], out_hbm.at[pl.ds((row0 + g * R) * LE, R * LE)], sems[b]
            ).wait()

        for b in range(2):
            compute(jnp.int32(b), outs[b])
            start_store(jnp.int32(b), b)

        @pl.loop(2, n_chunks, step=2)
        def _(g):
            for b in range(2):
                wait_store(g + b - 2, b)
                compute(g + b, outs[b])
                start_store(g + b, b)

        for b in range(2):
            wait_store(jnp.int32(n_chunks - 2 + b), b)

    return sc_kernel


def kernel(input, emb_table):
    B = input.shape[0]
    out = _make_sc_kernel(B)(input, emb_table.reshape(-1))
    return out.reshape(B, 1, LE)


# speedup vs baseline: 29.2104x; 1.0242x over previous
import functools

import jax
import jax.numpy as jnp
from jax import lax
from jax.experimental import pallas as pl
from jax.experimental.pallas import tpu as pltpu
from jax.experimental.pallas import tpu_sc as plsc

L = 100
E = 64
LE = L * E
LANES = 16
ROWS_PER_CHUNK = 8


@functools.lru_cache(maxsize=None)
def _make_sc_kernel(B: int):
    info = plsc.get_sparse_core_info()
    nw = info.num_cores * info.num_subcores
    rows_per_w = B // nw
    R = ROWS_PER_CHUNK
    n_chunks = rows_per_w // R
    mesh = plsc.VectorSubcoreMesh(core_axis_name="c", subcore_axis_name="s")

    @functools.partial(
        pl.kernel,
        mesh=mesh,
        out_type=jax.ShapeDtypeStruct((B * LE,), jnp.float32),
        scratch_types=[
            pltpu.VMEM((LE,), jnp.float32),
            pltpu.VMEM((rows_per_w, L), jnp.float32),
            pltpu.VMEM((R * LE,), jnp.float32),
            pltpu.VMEM((R * LE,), jnp.float32),
            pltpu.SemaphoreType.DMA,
            pltpu.SemaphoreType.DMA,
        ],
        compiler_params=pltpu.CompilerParams(needs_layout_passes=False),
    )
    def sc_kernel(in_hbm, emb_hbm, out_hbm, emb_v, in_v, out0, out1, sem0, sem1):
        c = lax.axis_index("c")
        s = lax.axis_index("s")
        wid = s * info.num_cores + c
        row0 = wid * rows_per_w
        pltpu.sync_copy(emb_hbm, emb_v)
        pltpu.sync_copy(in_hbm.at[pl.ds(row0, rows_per_w)], in_v)

        outs = (out0, out1)
        sems = (sem0, sem1)

        def compute(g, out_v):
            @plsc.parallel_loop(0, L, unroll=1)
            def _(i):
                evs = [
                    emb_v[pl.ds(i * E + kk * LANES, LANES)]
                    for kk in range(E // LANES)
                ]
                for r in range(R):
                    ridx = jnp.full((LANES,), g * R + r, jnp.int32)
                    iidx = jnp.full((LANES,), 0, jnp.int32) + i
                    sval = plsc.load_gather(in_v, [ridx, iidx])
                    for kk in range(E // LANES):
                        out_v[pl.ds(r * LE + i * E + kk * LANES, LANES)] = (
                            sval * evs[kk]
                        )

        def start_store(g, b):
            pltpu.async_copy(
                outs[b], out_hbm.at[pl.ds((row0 + g * R) * LE, R * LE)], sems[b]
            )

        def wait_store(g, b):
            pltpu.make_async_copy(
                outs[b], out_hbm.at[pl.ds((row0 + g * R) * LE, R * LE)], sems[b]
            ).wait()

        @pl.loop(0, n_chunks, step=2)
        def _(g):
            for b in range(2):
                @pl.when(g + b >= 2)
                def _():
                    wait_store(g + b - 2, b)

                compute(g + b, outs[b])
                start_store(g + b, b)

        for b in range(2):
            wait_store(jnp.int32(n_chunks - 2 + b), b)

    return sc_kernel


def kernel(input, emb_table):
    B = input.shape[0]
    out = _make_sc_kernel(B)(input, emb_table.reshape(-1))
    return out.reshape(B, 1, LE)
